# baseline (speedup 1.0000x reference)
import functools

import jax
import jax.numpy as jnp
from jax import lax
from jax.experimental import pallas as pl
from jax.experimental.pallas import tpu as pltpu
from jax.experimental.pallas import tpu_sc as plsc

N_NODES = 10000
N_EDGES = 320000
D_IN = 128
HID = 256
N_CLS = 40
W_PAD = 48

N_PAD = 10240

NC = 2
NS = 16
NW = NC * NS
EROW = 128
ROWS_PAD = 2560
E_PAD = ROWS_PAD * EROW
RPT = ROWS_PAD // NW
NB = 3
TILE_ROWS = N_PAD // NS


IDXC = 40


def _make_prop(width, gather):
  nb = 2 if width >= 96 else 4
  mesh = plsc.VectorSubcoreMesh(
      core_axis_name="c", subcore_axis_name="s", num_cores=NC, num_subcores=NS)

  @functools.partial(
      pl.kernel,
      out_type=jax.ShapeDtypeStruct((NC, N_PAD, width), jnp.float32),
      mesh=mesh,
      scratch_types=[
          pltpu.VMEM((IDXC if gather else 1, EROW), jnp.int32),
          pltpu.VMEM((IDXC, EROW), jnp.int32),
          pltpu.VMEM((nb, EROW, width), jnp.float32),
          pltpu.VMEM_SHARED((N_PAD, width), jnp.float32),
          pltpu.SemaphoreType.DMA((nb,)),
          pltpu.SemaphoreType.DMA((nb,)),
      ],
      compiler_params=pltpu.CompilerParams(use_tc_tiling_on_sc=False),
  )
  def prop(x_hbm, src_hbm, dst_hbm, zeros_hbm, out_hbm, idx_s, idx_d, rows,
           acc, gsem, ssem):
    c = lax.axis_index("c")
    s = lax.axis_index("s")
    wid = s * NC + c
    pltpu.sync_copy(zeros_hbm.at[pl.ds(s * TILE_ROWS, TILE_ROWS)],
                    acc.at[pl.ds(s * TILE_ROWS, TILE_ROWS)])
    if not gather:
      pltpu.sync_copy(x_hbm, rows.at[0])
    plsc.subcore_barrier()
    base = wid * RPT

    for phase in range(RPT // IDXC):
      pbase = base + phase * IDXC
      pltpu.sync_copy(dst_hbm.at[pl.ds(pbase, IDXC)], idx_d)
      if gather:
        pltpu.sync_copy(src_hbm.at[pl.ds(pbase, IDXC)], idx_s)

        @pl.loop(0, IDXC)
        def _(r):
          pltpu.async_copy(x_hbm.at[idx_s.at[r]], rows.at[0],
                           gsem.at[0]).wait()
          pltpu.sync_copy(rows.at[0], acc.at[idx_d.at[r]], add=True)
      else:
        @pl.loop(0, IDXC, step=nb)
        def _(r0):
          for b in range(nb):
            pltpu.async_copy(
                rows.at[0], acc.at[idx_d.at[r0 + b]], ssem.at[b], add=True)
          for b in range(nb):
            pltpu.make_async_copy(
                rows.at[0], acc.at[idx_d.at[0]], ssem.at[b]).wait()

    plsc.subcore_barrier()
    pltpu.sync_copy(acc.at[pl.ds(s * TILE_ROWS, TILE_ROWS)],
                    out_hbm.at[c, pl.ds(s * TILE_ROWS, TILE_ROWS)])

  return prop


_prop = functools.lru_cache(maxsize=None)(_make_prop)


def _dinv_body(p_ref, o_ref):
  deg = jnp.sum(p_ref[...], axis=0, keepdims=True)
  o_ref[...] = lax.rsqrt(jnp.maximum(deg, 1.0))


def _pre_body(feat_ref, dinv_ref, w1a_ref, b1_ref, xs_ref, z1_ref):
  f = feat_ref[...]
  xs_ref[...] = f * dinv_ref[...]
  z1_ref[...] = (
      jnp.dot(f, w1a_ref[...], preferred_element_type=jnp.float32)
      + b1_ref[...])


def _mid_body(z1_ref, a0_ref, a1_ref, dinv_ref, w1b_ref, w2_ref, b2_ref,
              y2a_ref, y2b_ref):
  d = dinv_ref[...]
  agg = (a0_ref[...] + a1_ref[...]) * d
  h = jnp.maximum(
      z1_ref[...]
      - jnp.dot(agg, w1b_ref[...], preferred_element_type=jnp.float32), 0.0)
  y = jnp.dot(h, w2_ref[...], preferred_element_type=jnp.float32)
  y2a_ref[...] = y[:, :N_CLS] + b2_ref[...]
  yb = y[:, N_CLS:] * d
  pad = jnp.zeros((yb.shape[0], W_PAD - N_CLS), jnp.float32)
  y2b_ref[...] = jnp.concatenate([yb, pad], axis=1)


def _fin_body(y2a_ref, q0_ref, q1_ref, dinv_ref, o_ref):
  q = (q0_ref[...] + q1_ref[...])[:, :N_CLS]
  o = y2a_ref[...] - q * dinv_ref[...]
  m = jnp.max(o, axis=1, keepdims=True)
  lse = jnp.log(jnp.sum(jnp.exp(o - m), axis=1, keepdims=True)) + m
  o_ref[...] = o - lse


_R = 1000
_GRID = (N_NODES // _R,)


def _rows(w):
  return pl.BlockSpec((_R, w), lambda i: (i, 0))


def _full(a, b):
  return pl.BlockSpec((a, b), lambda i: (0, 0))


_dinv_call = pl.pallas_call(
    _dinv_body,
    out_shape=jax.ShapeDtypeStruct((1, N_NODES), jnp.float32),
)

_pre_call = pl.pallas_call(
    _pre_body,
    grid=_GRID,
    in_specs=[_rows(D_IN), _rows(1), _full(D_IN, HID), _full(1, HID)],
    out_specs=[_rows(D_IN), _rows(HID)],
    out_shape=[
        jax.ShapeDtypeStruct((N_NODES, D_IN), jnp.float32),
        jax.ShapeDtypeStruct((N_NODES, HID), jnp.float32),
    ],
)

_mid_call = pl.pallas_call(
    _mid_body,
    grid=_GRID,
    in_specs=[
        _rows(HID), _rows(D_IN), _rows(D_IN), _rows(1),
        _full(D_IN, HID), _full(HID, 2 * N_CLS), _full(1, N_CLS),
    ],
    out_specs=[_rows(N_CLS), _rows(W_PAD)],
    out_shape=[
        jax.ShapeDtypeStruct((N_NODES, N_CLS), jnp.float32),
        jax.ShapeDtypeStruct((N_NODES, W_PAD), jnp.float32),
    ],
)

_fin_call = pl.pallas_call(
    _fin_body,
    grid=_GRID,
    in_specs=[_rows(N_CLS), _rows(W_PAD), _rows(W_PAD), _rows(1)],
    out_specs=_rows(N_CLS),
    out_shape=jax.ShapeDtypeStruct((N_NODES, N_CLS), jnp.float32),
)


@jax.jit
def kernel(feat, edge_index, W1, b1, W2, b2):
  n_pad_e = E_PAD - N_EDGES
  src2d = jnp.concatenate(
      [edge_index[0], jnp.zeros((n_pad_e,), jnp.int32)]).reshape(ROWS_PAD,
                                                                 EROW)
  pad_dst = N_NODES + jnp.arange(n_pad_e, dtype=jnp.int32) % (N_PAD - N_NODES)
  dst2d = jnp.concatenate(
      [edge_index[1], pad_dst]).reshape(ROWS_PAD, EROW)
  ones_r = jnp.ones((EROW, 8), jnp.float32)
  zeros1 = jnp.zeros((N_PAD, 8), jnp.float32)
  zeros128 = jnp.zeros((N_PAD, D_IN), jnp.float32)
  zeros48 = jnp.zeros((N_PAD, W_PAD), jnp.float32)

  degp = _prop(8, False)(ones_r, src2d, dst2d, zeros1)[:, :N_NODES, 0]
  dinv = _dinv_call(degp).reshape(N_NODES, 1)
  xs, z1 = _pre_call(feat, dinv, W1[:D_IN], b1.reshape(1, HID))
  a = _prop(D_IN, True)(xs, src2d, dst2d, zeros128)[:, :N_NODES]
  w2cat = jnp.concatenate([W2[:HID], W2[HID:]], axis=1)
  y2a, y2b = _mid_call(z1, a[0], a[1], dinv, W1[D_IN:], w2cat,
                       b2.reshape(1, N_CLS))
  q = _prop(W_PAD, True)(y2b, src2d, dst2d, zeros48)[:, :N_NODES]
  return _fin_call(y2a, q[0], q[1], dinv)


# speedup vs baseline: 1.5621x; 1.5621x over previous
import functools

import jax
import jax.numpy as jnp
from jax import lax
from jax.experimental import pallas as pl
from jax.experimental.pallas import tpu as pltpu
from jax.experimental.pallas import tpu_sc as plsc

N_NODES = 10000
N_EDGES = 320000
D_IN = 128
HID = 256
N_CLS = 40
W_PAD = 48

N_PAD = 10240

NC = 2
NS = 16
NW = NC * NS
EROW = 128
ROWS_E = N_EDGES // EROW
ROWS_PAD = 2560
E_PAD = ROWS_PAD * EROW
RPT = ROWS_PAD // NW
NB = 3
TILE_ROWS = N_PAD // NS


IDXC = 40


def _make_prop(width, gather):
  nb = 4
  mesh = plsc.VectorSubcoreMesh(
      core_axis_name="c", subcore_axis_name="s", num_cores=NC, num_subcores=NS)

  if gather:
    @functools.partial(
        pl.kernel,
        out_type=jax.ShapeDtypeStruct((NC, N_PAD, width), jnp.float32),
        mesh=mesh,
        scratch_types=[
            pltpu.VMEM((EROW,), jnp.int32),
            pltpu.VMEM((EROW,), jnp.int32),
            pltpu.VMEM((EROW, width), jnp.float32),
            pltpu.VMEM_SHARED((N_PAD, width), jnp.float32),
            pltpu.SemaphoreType.DMA,
        ],
        compiler_params=pltpu.CompilerParams(use_tc_tiling_on_sc=False),
    )
    def prop(x_hbm, src_hbm, dst_hbm, zeros_hbm, out_hbm, idx_s, idx_d, rows,
             acc, sem):
      c = lax.axis_index("c")
      s = lax.axis_index("s")
      wid = s * NC + c
      pltpu.sync_copy(zeros_hbm.at[pl.ds(s * TILE_ROWS, TILE_ROWS)],
                      acc.at[pl.ds(s * TILE_ROWS, TILE_ROWS)])
      plsc.subcore_barrier()
      start = wid * ROWS_E // NW
      stop = (wid + 1) * ROWS_E // NW

      def body(r, carry):
        pltpu.sync_copy(src_hbm.at[pl.ds(r * EROW, EROW)], idx_s)
        pltpu.sync_copy(dst_hbm.at[pl.ds(r * EROW, EROW)], idx_d)
        pltpu.async_copy(x_hbm.at[idx_s], rows, sem).wait()
        pltpu.sync_copy(rows, acc.at[idx_d], add=True)
        return carry

      lax.fori_loop(start, stop, body, 0)
      plsc.subcore_barrier()
      pltpu.sync_copy(acc.at[pl.ds(s * TILE_ROWS, TILE_ROWS)],
                      out_hbm.at[c, pl.ds(s * TILE_ROWS, TILE_ROWS)])

    return prop

  @functools.partial(
      pl.kernel,
      out_type=jax.ShapeDtypeStruct((NC, N_PAD, width), jnp.float32),
      mesh=mesh,
      scratch_types=[
          pltpu.VMEM((IDXC, EROW), jnp.int32),
          pltpu.VMEM((EROW, width), jnp.float32),
          pltpu.VMEM_SHARED((N_PAD, width), jnp.float32),
          pltpu.SemaphoreType.DMA((nb,)),
      ],
      compiler_params=pltpu.CompilerParams(use_tc_tiling_on_sc=False),
  )
  def deg(x_hbm, dst_hbm, zeros_hbm, out_hbm, idx_d, rows, acc, ssem):
    c = lax.axis_index("c")
    s = lax.axis_index("s")
    wid = s * NC + c
    pltpu.sync_copy(zeros_hbm.at[pl.ds(s * TILE_ROWS, TILE_ROWS)],
                    acc.at[pl.ds(s * TILE_ROWS, TILE_ROWS)])
    pltpu.sync_copy(x_hbm, rows)
    plsc.subcore_barrier()
    base = wid * RPT

    for phase in range(RPT // IDXC):
      pltpu.sync_copy(dst_hbm.at[pl.ds(base + phase * IDXC, IDXC)], idx_d)

      @pl.loop(0, IDXC, step=nb)
      def _(r0):
        for b in range(nb):
          pltpu.async_copy(
              rows, acc.at[idx_d.at[r0 + b]], ssem.at[b], add=True)
        for b in range(nb):
          pltpu.make_async_copy(rows, acc.at[idx_d.at[0]], ssem.at[b]).wait()

    plsc.subcore_barrier()
    pltpu.sync_copy(acc.at[pl.ds(s * TILE_ROWS, TILE_ROWS)],
                    out_hbm.at[c, pl.ds(s * TILE_ROWS, TILE_ROWS)])

  return deg


_prop = functools.lru_cache(maxsize=None)(_make_prop)


def _dinv_body(p_ref, o_ref):
  deg = jnp.sum(p_ref[...], axis=0, keepdims=True)
  o_ref[...] = lax.rsqrt(jnp.maximum(deg, 1.0))


def _pre_body(feat_ref, dinv_ref, w1a_ref, b1_ref, xs_ref, z1_ref):
  f = feat_ref[...]
  xs_ref[...] = f * dinv_ref[...]
  z1_ref[...] = (
      jnp.dot(f, w1a_ref[...], preferred_element_type=jnp.float32)
      + b1_ref[...])


def _mid_body(z1_ref, a0_ref, a1_ref, dinv_ref, w1b_ref, w2_ref, b2_ref,
              y2a_ref, y2b_ref):
  d = dinv_ref[...]
  agg = (a0_ref[...] + a1_ref[...]) * d
  h = jnp.maximum(
      z1_ref[...]
      - jnp.dot(agg, w1b_ref[...], preferred_element_type=jnp.float32), 0.0)
  y = jnp.dot(h, w2_ref[...], preferred_element_type=jnp.float32)
  y2a_ref[...] = y[:, :N_CLS] + b2_ref[...]
  yb = y[:, N_CLS:] * d
  pad = jnp.zeros((yb.shape[0], W_PAD - N_CLS), jnp.float32)
  y2b_ref[...] = jnp.concatenate([yb, pad], axis=1)


def _fin_body(y2a_ref, q0_ref, q1_ref, dinv_ref, o_ref):
  q = (q0_ref[...] + q1_ref[...])[:, :N_CLS]
  o = y2a_ref[...] - q * dinv_ref[...]
  m = jnp.max(o, axis=1, keepdims=True)
  lse = jnp.log(jnp.sum(jnp.exp(o - m), axis=1, keepdims=True)) + m
  o_ref[...] = o - lse


_R = 1000
_GRID = (N_NODES // _R,)


def _rows(w):
  return pl.BlockSpec((_R, w), lambda i: (i, 0))


def _full(a, b):
  return pl.BlockSpec((a, b), lambda i: (0, 0))


_dinv_call = pl.pallas_call(
    _dinv_body,
    out_shape=jax.ShapeDtypeStruct((1, N_NODES), jnp.float32),
)

_pre_call = pl.pallas_call(
    _pre_body,
    grid=_GRID,
    in_specs=[_rows(D_IN), _rows(1), _full(D_IN, HID), _full(1, HID)],
    out_specs=[_rows(D_IN), _rows(HID)],
    out_shape=[
        jax.ShapeDtypeStruct((N_NODES, D_IN), jnp.float32),
        jax.ShapeDtypeStruct((N_NODES, HID), jnp.float32),
    ],
)

_mid_call = pl.pallas_call(
    _mid_body,
    grid=_GRID,
    in_specs=[
        _rows(HID), _rows(D_IN), _rows(D_IN), _rows(1),
        _full(D_IN, HID), _full(HID, 2 * N_CLS), _full(1, N_CLS),
    ],
    out_specs=[_rows(N_CLS), _rows(W_PAD)],
    out_shape=[
        jax.ShapeDtypeStruct((N_NODES, N_CLS), jnp.float32),
        jax.ShapeDtypeStruct((N_NODES, W_PAD), jnp.float32),
    ],
)

_fin_call = pl.pallas_call(
    _fin_body,
    grid=_GRID,
    in_specs=[_rows(N_CLS), _rows(W_PAD), _rows(W_PAD), _rows(1)],
    out_specs=_rows(N_CLS),
    out_shape=jax.ShapeDtypeStruct((N_NODES, N_CLS), jnp.float32),
)


@jax.jit
def kernel(feat, edge_index, W1, b1, W2, b2):
  src = edge_index[0]
  dst = edge_index[1]
  n_pad_e = E_PAD - N_EDGES
  pad_dst = N_NODES + jnp.arange(n_pad_e, dtype=jnp.int32) % (N_PAD - N_NODES)
  dst2d = jnp.concatenate([dst, pad_dst]).reshape(ROWS_PAD, EROW)
  ones_r = jnp.ones((EROW, 8), jnp.float32)
  zeros1 = jnp.zeros((N_PAD, 8), jnp.float32)
  zeros128 = jnp.zeros((N_PAD, D_IN), jnp.float32)
  zeros48 = jnp.zeros((N_PAD, W_PAD), jnp.float32)

  degp = _prop(8, False)(ones_r, dst2d, zeros1)[:, :N_NODES, 0]
  dinv = _dinv_call(degp).reshape(N_NODES, 1)
  xs, z1 = _pre_call(feat, dinv, W1[:D_IN], b1.reshape(1, HID))
  a = _prop(D_IN, True)(xs, src, dst, zeros128)[:, :N_NODES]
  w2cat = jnp.concatenate([W2[:HID], W2[HID:]], axis=1)
  y2a, y2b = _mid_call(z1, a[0], a[1], dinv, W1[D_IN:], w2cat,
                       b2.reshape(1, N_CLS))
  q = _prop(W_PAD, True)(y2b, src, dst, zeros48)[:, :N_NODES]
  return _fin_call(y2a, q[0], q[1], dinv)


# speedup vs baseline: 1.6013x; 1.0251x over previous
import functools

import jax
import jax.numpy as jnp
from jax import lax
from jax.experimental import pallas as pl
from jax.experimental.pallas import tpu as pltpu
from jax.experimental.pallas import tpu_sc as plsc

N_NODES = 10000
N_EDGES = 320000
D_IN = 128
HID = 256
N_CLS = 40
W_PAD = 48

N_PAD = 10240

NC = 2
NS = 16
NW = NC * NS
EROW = 128
ROWS_E = N_EDGES // EROW
ROWS_PAD = 2560
E_PAD = ROWS_PAD * EROW
RPT = ROWS_PAD // NW
NB = 3
TILE_ROWS = N_PAD // NS


IDXC = 40


def _make_prop(width, gather):
  nb = 4
  mesh = plsc.VectorSubcoreMesh(
      core_axis_name="c", subcore_axis_name="s", num_cores=NC, num_subcores=NS)

  if gather:
    @functools.partial(
        pl.kernel,
        out_type=jax.ShapeDtypeStruct((NC, N_PAD, width), jnp.float32),
        mesh=mesh,
        scratch_types=[
            pltpu.VMEM((EROW,), jnp.int32),
            pltpu.VMEM((EROW,), jnp.int32),
            pltpu.VMEM((EROW, width), jnp.float32),
            pltpu.VMEM_SHARED((N_PAD, width), jnp.float32),
            pltpu.SemaphoreType.DMA,
        ],
        compiler_params=pltpu.CompilerParams(use_tc_tiling_on_sc=False),
    )
    def prop(x_hbm, src_hbm, dst_hbm, zeros_hbm, out_hbm, idx_s, idx_d, rows,
             acc, sem):
      c = lax.axis_index("c")
      s = lax.axis_index("s")
      wid = s * NC + c
      pltpu.sync_copy(zeros_hbm.at[pl.ds(s * TILE_ROWS, TILE_ROWS)],
                      acc.at[pl.ds(s * TILE_ROWS, TILE_ROWS)])
      plsc.subcore_barrier()
      start = wid * ROWS_E // NW
      stop = (wid + 1) * ROWS_E // NW

      def body(r, carry):
        pltpu.sync_copy(src_hbm.at[pl.ds(r * EROW, EROW)], idx_s)
        pltpu.sync_copy(dst_hbm.at[pl.ds(r * EROW, EROW)], idx_d)
        pltpu.async_copy(x_hbm.at[idx_s], rows, sem).wait()
        pltpu.sync_copy(rows, acc.at[idx_d], add=True)
        return carry

      lax.fori_loop(start, stop, body, 0)
      plsc.subcore_barrier()
      pltpu.sync_copy(acc.at[pl.ds(s * TILE_ROWS, TILE_ROWS)],
                      out_hbm.at[c, pl.ds(s * TILE_ROWS, TILE_ROWS)])

    return prop

  @functools.partial(
      pl.kernel,
      out_type=jax.ShapeDtypeStruct((NC, N_PAD, width), jnp.float32),
      mesh=mesh,
      scratch_types=[
          pltpu.VMEM((IDXC, EROW), jnp.int32),
          pltpu.VMEM((EROW, width), jnp.float32),
          pltpu.VMEM_SHARED((N_PAD, width), jnp.float32),
          pltpu.SemaphoreType.DMA((nb,)),
      ],
      compiler_params=pltpu.CompilerParams(use_tc_tiling_on_sc=False),
  )
  def deg(x_hbm, dst_hbm, zeros_hbm, out_hbm, idx_d, rows, acc, ssem):
    c = lax.axis_index("c")
    s = lax.axis_index("s")
    wid = s * NC + c
    pltpu.sync_copy(zeros_hbm.at[pl.ds(s * TILE_ROWS, TILE_ROWS)],
                    acc.at[pl.ds(s * TILE_ROWS, TILE_ROWS)])
    pltpu.sync_copy(x_hbm, rows)
    plsc.subcore_barrier()
    base = wid * RPT

    for phase in range(RPT // IDXC):
      pltpu.sync_copy(dst_hbm.at[pl.ds(base + phase * IDXC, IDXC)], idx_d)

      @pl.loop(0, IDXC, step=nb)
      def _(r0):
        for b in range(nb):
          pltpu.async_copy(
              rows, acc.at[idx_d.at[r0 + b]], ssem.at[b], add=True)
        for b in range(nb):
          pltpu.make_async_copy(rows, acc.at[idx_d.at[0]], ssem.at[b]).wait()

    plsc.subcore_barrier()
    pltpu.sync_copy(acc.at[pl.ds(s * TILE_ROWS, TILE_ROWS)],
                    out_hbm.at[c, pl.ds(s * TILE_ROWS, TILE_ROWS)])

  return deg


_prop = functools.lru_cache(maxsize=None)(_make_prop)


def _dinv_body(p_ref, o_ref):
  deg = jnp.sum(p_ref[...], axis=0, keepdims=True)
  o_ref[...] = lax.rsqrt(jnp.maximum(deg, 1.0))


def _pre_body(feat_ref, dinv_ref, w1a_ref, b1_ref, xs_ref, z1_ref):
  f = feat_ref[...]
  xs_ref[...] = f * dinv_ref[...]
  z1_ref[...] = (
      jnp.dot(f, w1a_ref[...], preferred_element_type=jnp.float32)
      + b1_ref[...])


def _mid_body(z1_ref, a0_ref, a1_ref, dinv_ref, w1b_ref, w2_ref, b2_ref,
              y2a_ref, y2b_ref):
  d = dinv_ref[...]
  agg = (a0_ref[0] + a1_ref[0]) * d
  h = jnp.maximum(
      z1_ref[...]
      - jnp.dot(agg, w1b_ref[...], preferred_element_type=jnp.float32), 0.0)
  y = jnp.dot(h, w2_ref[...], preferred_element_type=jnp.float32)
  y2a_ref[...] = y[:, :N_CLS] + b2_ref[...]
  yb = y[:, N_CLS:] * d
  pad = jnp.zeros((yb.shape[0], W_PAD - N_CLS), jnp.float32)
  y2b_ref[...] = jnp.concatenate([yb, pad], axis=1)


def _fin_body(y2a_ref, q0_ref, q1_ref, dinv_ref, o_ref):
  q = (q0_ref[0] + q1_ref[0])[:, :N_CLS]
  o = y2a_ref[...] - q * dinv_ref[...]
  m = jnp.max(o, axis=1, keepdims=True)
  lse = jnp.log(jnp.sum(jnp.exp(o - m), axis=1, keepdims=True)) + m
  o_ref[...] = o - lse


_R = 1000
_GRID = (N_NODES // _R,)


def _rows(w):
  return pl.BlockSpec((_R, w), lambda i: (i, 0))


def _full(a, b):
  return pl.BlockSpec((a, b), lambda i: (0, 0))


def _part(w, k):
  return pl.BlockSpec((1, _R, w), lambda i, _k=k: (_k, i, 0))


_dinv_call = pl.pallas_call(
    _dinv_body,
    out_shape=jax.ShapeDtypeStruct((1, N_NODES), jnp.float32),
)

_pre_call = pl.pallas_call(
    _pre_body,
    grid=_GRID,
    in_specs=[_rows(D_IN), _rows(1), _full(D_IN, HID), _full(1, HID)],
    out_specs=[_rows(D_IN), _rows(HID)],
    out_shape=[
        jax.ShapeDtypeStruct((N_NODES, D_IN), jnp.float32),
        jax.ShapeDtypeStruct((N_NODES, HID), jnp.float32),
    ],
)

_mid_call = pl.pallas_call(
    _mid_body,
    grid=_GRID,
    in_specs=[
        _rows(HID), _part(D_IN, 0), _part(D_IN, 1), _rows(1),
        _full(D_IN, HID), _full(HID, 2 * N_CLS), _full(1, N_CLS),
    ],
    out_specs=[_rows(N_CLS), _rows(W_PAD)],
    out_shape=[
        jax.ShapeDtypeStruct((N_NODES, N_CLS), jnp.float32),
        jax.ShapeDtypeStruct((N_NODES, W_PAD), jnp.float32),
    ],
)

_fin_call = pl.pallas_call(
    _fin_body,
    grid=_GRID,
    in_specs=[_rows(N_CLS), _part(W_PAD, 0), _part(W_PAD, 1), _rows(1)],
    out_specs=_rows(N_CLS),
    out_shape=jax.ShapeDtypeStruct((N_NODES, N_CLS), jnp.float32),
)


@jax.jit
def kernel(feat, edge_index, W1, b1, W2, b2):
  src = edge_index[0]
  dst = edge_index[1]
  n_pad_e = E_PAD - N_EDGES
  pad_dst = N_NODES + jnp.arange(n_pad_e, dtype=jnp.int32) % (N_PAD - N_NODES)
  dst2d = jnp.concatenate([dst, pad_dst]).reshape(ROWS_PAD, EROW)
  ones_r = jnp.ones((EROW, 8), jnp.float32)
  zeros1 = jnp.zeros((N_PAD, 8), jnp.float32)
  zeros128 = jnp.zeros((N_PAD, D_IN), jnp.float32)
  zeros48 = jnp.zeros((N_PAD, W_PAD), jnp.float32)

  degp = _prop(8, False)(ones_r, dst2d, zeros1)[:, :N_NODES, 0]
  dinv = _dinv_call(degp).reshape(N_NODES, 1)
  xs, z1 = _pre_call(feat, dinv, W1[:D_IN], b1.reshape(1, HID))
  a = _prop(D_IN, True)(xs, src, dst, zeros128)
  w2cat = jnp.concatenate([W2[:HID], W2[HID:]], axis=1)
  y2a, y2b = _mid_call(z1, a, a, dinv, W1[D_IN:], w2cat,
                       b2.reshape(1, N_CLS))
  q = _prop(W_PAD, True)(y2b, src, dst, zeros48)
  return _fin_call(y2a, q, q, dinv)
